# baseline (speedup 1.0000x reference)
import functools

import jax
import jax.numpy as jnp
from jax import lax
from jax.experimental import pallas as pl
from jax.experimental.pallas import tpu as pltpu
from jax.experimental.pallas import tpu_sc as plsc

B = 16384
F = 32
NC = 2
NS = 16
NW = NC * NS
BPW = B // NW
CHUNK = 128
NCHUNK = BPW // CHUNK
GROUPS = BPW // 16


def _mf_body(uid_hbm, iid_hbm, ue_hbm, ub_hbm, ie_hbm, ib_hbm, out_hbm,
             uidx, iidx, urows, irows, ubv, ibv, outv, sem):
    wid = lax.axis_index("s") * NC + lax.axis_index("c")
    base = wid * BPW

    pltpu.sync_copy(uid_hbm.at[wid], uidx)
    pltpu.sync_copy(iid_hbm.at[wid], iidx)

    copies = []
    for j in range(NCHUNK):
        sl = pl.ds(j * CHUNK, CHUNK)
        copies.append(pltpu.async_copy(ue_hbm.at[uidx.at[j]], urows.at[sl], sem))
        copies.append(pltpu.async_copy(ie_hbm.at[iidx.at[j]], irows.at[sl], sem))
        copies.append(pltpu.async_copy(ub_hbm.at[uidx.at[j]], ubv.at[sl], sem))
        copies.append(pltpu.async_copy(ib_hbm.at[iidx.at[j]], ibv.at[sl], sem))
    for cp in copies:
        cp.wait()

    def group(g, carry):
        row0 = g * 16
        rows = row0 + lax.iota(jnp.int32, 16)
        acc = ubv[pl.ds(row0, 16)] + ibv[pl.ds(row0, 16)]
        for f in range(F):
            cols = jnp.zeros((16,), jnp.int32) + f
            uv = plsc.load_gather(urows, [rows, cols])
            iv = plsc.load_gather(irows, [rows, cols])
            acc = acc + uv * iv
        outv[pl.ds(row0, 16)] = acc
        return carry

    lax.fori_loop(0, GROUPS, group, 0)
    pltpu.sync_copy(outv, out_hbm.at[pl.ds(base, BPW)])


@functools.partial(jax.jit, donate_argnums=())
def _mf(uid, iid, ue, ub, ie, ib):
    mesh = plsc.VectorSubcoreMesh(core_axis_name="c", subcore_axis_name="s")
    run = pl.kernel(
        _mf_body,
        mesh=mesh,
        compiler_params=pltpu.CompilerParams(
            needs_layout_passes=False, use_tc_tiling_on_sc=False),
        out_type=jax.ShapeDtypeStruct((B,), jnp.float32),
        scratch_types=[
            pltpu.VMEM((NCHUNK, CHUNK), jnp.int32),
            pltpu.VMEM((NCHUNK, CHUNK), jnp.int32),
            pltpu.VMEM((BPW, F), jnp.float32),
            pltpu.VMEM((BPW, F), jnp.float32),
            pltpu.VMEM((BPW,), jnp.float32),
            pltpu.VMEM((BPW,), jnp.float32),
            pltpu.VMEM((BPW,), jnp.float32),
            pltpu.SemaphoreType.DMA,
        ],
    )
    return run(uid, iid, ue, ub, ie, ib)


def kernel(user_ids, item_ids, user_embed, user_bias_embed, item_embed,
           item_bias_embed):
    uid = user_ids.astype(jnp.int32).reshape(NW, NCHUNK, CHUNK)
    iid = item_ids.astype(jnp.int32).reshape(NW, NCHUNK, CHUNK)
    ub = user_bias_embed.reshape(-1)
    ib = item_bias_embed.reshape(-1)
    out = _mf(uid, iid, user_embed, ub, item_embed, ib)
    return out.reshape(B, 1)


# speedup vs baseline: 1.0020x; 1.0020x over previous
import functools

import jax
import jax.numpy as jnp
from jax import lax
from jax.experimental import pallas as pl
from jax.experimental.pallas import tpu as pltpu
from jax.experimental.pallas import tpu_sc as plsc

B = 16384
F = 32
NC = 2
NS = 16
NW = NC * NS
BPW = B // NW
CHUNK = 128
NCHUNK = BPW // CHUNK
GROUPS = BPW // 16


def _mf_body(uid_hbm, iid_hbm, ue_hbm, ub_hbm, ie_hbm, ib_hbm, out_hbm,
             uidx, iidx, urows, irows, ubv, ibv, outv, sem):
    wid = lax.axis_index("s") * NC + lax.axis_index("c")
    base = wid * BPW

    pltpu.sync_copy(uid_hbm.at[wid], uidx)
    pltpu.sync_copy(iid_hbm.at[wid], iidx)

    copies = []
    for j in range(NCHUNK):
        sl = pl.ds(j * CHUNK, CHUNK)
        copies.append(pltpu.async_copy(ue_hbm.at[uidx.at[j]], urows.at[sl], sem))
        copies.append(pltpu.async_copy(ie_hbm.at[iidx.at[j]], irows.at[sl], sem))
        copies.append(pltpu.async_copy(ub_hbm.at[uidx.at[j]], ubv.at[sl], sem))
        copies.append(pltpu.async_copy(ib_hbm.at[iidx.at[j]], ibv.at[sl], sem))
    for cp in copies:
        cp.wait()

    def group(g, carry):
        row0 = g * 16
        rows = row0 + lax.iota(jnp.int32, 16)
        acc = ubv[pl.ds(row0, 16)] + ibv[pl.ds(row0, 16)]
        for f in range(F):
            cols = jnp.zeros((16,), jnp.int32) + f
            uv = plsc.load_gather(urows, [rows, cols])
            iv = plsc.load_gather(irows, [rows, cols])
            acc = acc + uv * iv
        outv[pl.ds(row0, 16)] = acc
        return carry

    lax.fori_loop(0, GROUPS, group, 0)
    pltpu.sync_copy(outv, out_hbm.at[pl.ds(base, BPW)])


@functools.partial(jax.jit, donate_argnums=())
def _mf(uid, iid, ue, ub, ie, ib):
    mesh = plsc.VectorSubcoreMesh(core_axis_name="c", subcore_axis_name="s")
    run = pl.kernel(
        _mf_body,
        mesh=mesh,
        compiler_params=pltpu.CompilerParams(
            needs_layout_passes=False, use_tc_tiling_on_sc=False),
        out_type=jax.ShapeDtypeStruct((B,), jnp.float32),
        scratch_types=[
            pltpu.VMEM((NCHUNK, CHUNK), jnp.int32),
            pltpu.VMEM((NCHUNK, CHUNK), jnp.int32),
            pltpu.VMEM((BPW, F), jnp.float32),
            pltpu.VMEM((BPW, F), jnp.float32),
            pltpu.VMEM((BPW,), jnp.float32),
            pltpu.VMEM((BPW,), jnp.float32),
            pltpu.VMEM((BPW,), jnp.float32),
            pltpu.SemaphoreType.DMA,
        ],
    )
    return run(uid, iid, ue, ub, ie, ib)


def kernel(user_ids, item_ids, user_embed, user_bias_embed, item_embed,
           item_bias_embed):
    uid = user_ids.astype(jnp.int32).reshape(NW, NCHUNK, CHUNK)
    iid = item_ids.astype(jnp.int32).reshape(NW, NCHUNK, CHUNK)
    zero = lax.optimization_barrier(jnp.zeros((), jnp.float32))
    ub = (user_bias_embed - zero).reshape(-1)
    ib = (item_bias_embed - zero).reshape(-1)
    out = _mf(uid, iid, user_embed, ub, item_embed, ib)
    return out.reshape(B, 1)
